# baseline (speedup 1.0000x reference)
import functools

import jax
import jax.numpy as jnp
from jax import lax
from jax.experimental import pallas as pl
from jax.experimental.pallas import tpu as pltpu
from jax.experimental.pallas import tpu_sc as plsc


def _build(B, D, NC, NS):
    NW = NC * NS
    b_per_w = B // NW
    mesh = plsc.VectorSubcoreMesh(core_axis_name="c", subcore_axis_name="s")

    @functools.partial(
        pl.kernel,
        mesh=mesh,
        compiler_params=pltpu.CompilerParams(use_tc_tiling_on_sc=False),
        out_type=jax.ShapeDtypeStruct((B, 2 * D), jnp.float32),
        scratch_types=[
            pltpu.VMEM((b_per_w,), jnp.int32),
            pltpu.VMEM((b_per_w,), jnp.int32),
            pltpu.VMEM((b_per_w, 2 * D), jnp.float32),
            pltpu.SemaphoreType.DMA,
        ],
    )
    def k(uid_hbm, lid_hbm, tu_hbm, tl_hbm, out_hbm,
          idx_u, idx_l, asm, sem):
        wid = lax.axis_index("s") * NC + lax.axis_index("c")
        base = wid * b_per_w
        pltpu.sync_copy(uid_hbm.at[pl.ds(base, b_per_w)], idx_u)
        pltpu.sync_copy(lid_hbm.at[pl.ds(base, b_per_w)], idx_l)

        def body(g, _):
            vu = idx_u[pl.ds(g * 16, 16)]
            vl = idx_l[pl.ds(g * 16, 16)]
            for i in range(16):
                r = g * 16 + i
                pltpu.async_copy(
                    tu_hbm.at[pl.ds(vu[i], 1)],
                    asm.at[pl.ds(r, 1), pl.ds(0, D)], sem)
                pltpu.async_copy(
                    tl_hbm.at[pl.ds(vl[i], 1)],
                    asm.at[pl.ds(r, 1), pl.ds(D, D)], sem)
            return _

        lax.fori_loop(0, b_per_w // 16, body, 0)

        def drain(r, _):
            pltpu.make_async_copy(
                tu_hbm.at[pl.ds(0, 1)],
                asm.at[pl.ds(0, 1), pl.ds(0, D)], sem).wait()
            pltpu.make_async_copy(
                tu_hbm.at[pl.ds(0, 1)],
                asm.at[pl.ds(0, 1), pl.ds(D, D)], sem).wait()
            return _

        lax.fori_loop(0, b_per_w, drain, 0)
        pltpu.sync_copy(asm, out_hbm.at[pl.ds(base, b_per_w)])

    return k


def kernel(user_fea, table_user, table_location):
    B, _ = user_fea.shape
    D = table_user.shape[1]
    info = plsc.get_sparse_core_info()
    NC, NS = info.num_cores, info.num_subcores
    uid = user_fea[:, 0].astype(jnp.int32)
    lid = user_fea[:, 1].astype(jnp.int32)
    k = _build(B, D, NC, NS)
    return k(uid, lid, table_user, table_location)


# speedup vs baseline: 1.0002x; 1.0002x over previous
import functools

import jax
import jax.numpy as jnp
from jax import lax
from jax.experimental import pallas as pl
from jax.experimental.pallas import tpu as pltpu
from jax.experimental.pallas import tpu_sc as plsc


def _build(B, D, NC, NS):
    NW = NC * NS
    b_per_w = B // NW
    mesh = plsc.VectorSubcoreMesh(core_axis_name="c", subcore_axis_name="s")

    @functools.partial(
        pl.kernel,
        mesh=mesh,
        compiler_params=pltpu.CompilerParams(use_tc_tiling_on_sc=False),
        out_type=jax.ShapeDtypeStruct((B, 2 * D), jnp.float32),
        scratch_types=[
            pltpu.VMEM((b_per_w,), jnp.int32),
            pltpu.VMEM((b_per_w,), jnp.int32),
            pltpu.VMEM((b_per_w, 2 * D), jnp.float32),
            pltpu.SemaphoreType.DMA,
        ],
    )
    def k(uid_hbm, lid_hbm, tu_hbm, tl_hbm, out_hbm,
          idx_u, idx_l, asm, sem):
        wid = lax.axis_index("s") * NC + lax.axis_index("c")
        base = wid * b_per_w
        pltpu.sync_copy(uid_hbm.at[pl.ds(base, b_per_w)], idx_u)
        pltpu.sync_copy(lid_hbm.at[pl.ds(base, b_per_w)], idx_l)

        def fire(g, _):
            vu = idx_u[pl.ds(g * 16, 16)]
            vl = idx_l[pl.ds(g * 16, 16)]
            for i in range(16):
                r = g * 16 + i
                pltpu.async_copy(
                    tu_hbm.at[pl.ds(vu[i], 1)],
                    asm.at[pl.ds(r, 1), pl.ds(0, D)], sem)
                pltpu.async_copy(
                    tl_hbm.at[pl.ds(vl[i], 1)],
                    asm.at[pl.ds(r, 1), pl.ds(D, D)], sem)
            return _

        lax.fori_loop(0, b_per_w // 16, fire, 0)

        def drain(r, _):
            pltpu.make_async_copy(
                tu_hbm.at[pl.ds(0, 1)],
                asm.at[pl.ds(0, 1), pl.ds(0, D)], sem).wait()
            pltpu.make_async_copy(
                tu_hbm.at[pl.ds(0, 1)],
                asm.at[pl.ds(0, 1), pl.ds(D, D)], sem).wait()
            return _

        lax.fori_loop(0, b_per_w, drain, 0)
        pltpu.sync_copy(asm, out_hbm.at[pl.ds(base, b_per_w)])

    return k


def kernel(user_fea, table_user, table_location):
    B, _ = user_fea.shape
    D = table_user.shape[1]
    info = plsc.get_sparse_core_info()
    NC, NS = info.num_cores, info.num_subcores
    uid = user_fea[:, 0].astype(jnp.int32)
    lid = user_fea[:, 1].astype(jnp.int32)
    k = _build(B, D, NC, NS)
    return k(uid, lid, table_user, table_location)


# speedup vs baseline: 3.5922x; 3.5917x over previous
import functools

import jax
import jax.numpy as jnp
from jax import lax
from jax.experimental import pallas as pl
from jax.experimental.pallas import tpu as pltpu
from jax.experimental.pallas import tpu_sc as plsc

_DEPTH = 4


def _build(B, D, NC, NS):
    NW = NC * NS
    b_per_w = B // NW
    mesh = plsc.VectorSubcoreMesh(core_axis_name="c", subcore_axis_name="s")

    @functools.partial(
        pl.kernel,
        mesh=mesh,
        compiler_params=pltpu.CompilerParams(
            use_tc_tiling_on_sc=True, needs_layout_passes=False),
        out_type=jax.ShapeDtypeStruct((B, 4 * D), jnp.float32),
        scratch_types=[
            pltpu.VMEM((b_per_w + 16,), jnp.int32),
            pltpu.VMEM((b_per_w + 16,), jnp.int32),
            pltpu.VMEM((_DEPTH, 2, 32, 128), jnp.float32),
            pltpu.VMEM((b_per_w, 4 * D), jnp.float32),
        ] + [pltpu.SemaphoreType.DMA] * _DEPTH,
    )
    def k(uid_hbm, lid_hbm, tu_hbm, tl_hbm, out_hbm,
          idx_u, idx_l, tiles, asm, *sems):
        wid = lax.axis_index("s") * NC + lax.axis_index("c")
        base = wid * b_per_w
        pltpu.sync_copy(uid_hbm.at[pl.ds(base, b_per_w)],
                        idx_u.at[pl.ds(0, b_per_w)])
        pltpu.sync_copy(lid_hbm.at[pl.ds(base, b_per_w)],
                        idx_l.at[pl.ds(0, b_per_w)])

        lanes = lax.iota(jnp.int32, 16)

        def fire(r, buf):
            iu = idx_u[pl.ds(r, 16)][0]
            il = idx_l[pl.ds(r, 16)][0]
            ou = pl.multiple_of((iu >> 7) * 128, 128)
            ol = pl.multiple_of((il >> 7) * 128, 128)
            pltpu.async_copy(
                tu_hbm.at[pl.ds(0, 32), pl.ds(ou, 128)],
                tiles.at[buf, 0], sems[buf])
            pltpu.async_copy(
                tl_hbm.at[pl.ds(0, 32), pl.ds(ol, 128)],
                tiles.at[buf, 1], sems[buf])

        def drain(buf):
            for _ in range(2):
                pltpu.make_async_copy(
                    tu_hbm.at[pl.ds(0, 32), pl.ds(0, 128)],
                    tiles.at[buf, 0], sems[buf]).wait()

        def extract(r, buf):
            iu = idx_u[pl.ds(r, 16)][0]
            il = idx_l[pl.ds(r, 16)][0]
            imu = jnp.full((16,), iu & 127, jnp.int32)
            iml = jnp.full((16,), il & 127, jnp.int32)
            asm[r, pl.ds(0, 16)] = plsc.load_gather(
                tiles.at[buf, 0], [lanes, imu])
            asm[r, pl.ds(16, 16)] = plsc.load_gather(
                tiles.at[buf, 0], [lanes + 16, imu])
            asm[r, pl.ds(32, 16)] = plsc.load_gather(
                tiles.at[buf, 1], [lanes, iml])
            asm[r, pl.ds(48, 16)] = plsc.load_gather(
                tiles.at[buf, 1], [lanes + 16, iml])

        for p in range(_DEPTH - 1):
            fire(p, p)

        def body(s, _):
            for p in range(_DEPTH):
                r = s * _DEPTH + p
                fire(r + _DEPTH - 1, (p + _DEPTH - 1) % _DEPTH)
                drain(p)
                extract(r, p)
            return _

        n_steps = (b_per_w - (_DEPTH - 1)) // _DEPTH
        lax.fori_loop(0, n_steps, body, 0)
        tail = n_steps * _DEPTH
        fire(b_per_w - 1, (b_per_w - 1) % _DEPTH)
        for r in range(tail, b_per_w):
            drain(r % _DEPTH)
            extract(r, r % _DEPTH)
        pltpu.sync_copy(asm, out_hbm.at[pl.ds(base, b_per_w)])

    return k


def kernel(user_fea, table_user, table_location):
    B, _ = user_fea.shape
    D = table_user.shape[1]
    info = plsc.get_sparse_core_info()
    NC, NS = info.num_cores, info.num_subcores
    uid = user_fea[:, 0].astype(jnp.int32)
    lid = user_fea[:, 1].astype(jnp.int32)
    k = _build(B, D, NC, NS)
    out = k(uid, lid, table_user.T, table_location.T)
    return out[:, :2 * D]


# speedup vs baseline: 3.9733x; 1.1061x over previous
import functools

import jax
import jax.numpy as jnp
from jax import lax
from jax.experimental import pallas as pl
from jax.experimental.pallas import tpu as pltpu
from jax.experimental.pallas import tpu_sc as plsc

_DEPTH = 6


def _build(B, D, NC, NS):
    NW = NC * NS
    b_per_w = B // NW
    mesh = plsc.VectorSubcoreMesh(core_axis_name="c", subcore_axis_name="s")

    @functools.partial(
        pl.kernel,
        mesh=mesh,
        compiler_params=pltpu.CompilerParams(
            use_tc_tiling_on_sc=True, needs_layout_passes=False),
        out_type=jax.ShapeDtypeStruct((B, 4 * D), jnp.float32),
        scratch_types=[
            pltpu.VMEM((b_per_w + 16,), jnp.int32),
            pltpu.VMEM((b_per_w + 16,), jnp.int32),
            pltpu.VMEM((_DEPTH, 2, 32, 128), jnp.float32),
            pltpu.VMEM((b_per_w, 4 * D), jnp.float32),
        ] + [pltpu.SemaphoreType.DMA] * _DEPTH,
    )
    def k(uid_hbm, lid_hbm, tu_hbm, tl_hbm, out_hbm,
          idx_u, idx_l, tiles, asm, *sems):
        wid = lax.axis_index("s") * NC + lax.axis_index("c")
        base = wid * b_per_w
        pltpu.sync_copy(uid_hbm.at[pl.ds(base, b_per_w)],
                        idx_u.at[pl.ds(0, b_per_w)])
        pltpu.sync_copy(lid_hbm.at[pl.ds(base, b_per_w)],
                        idx_l.at[pl.ds(0, b_per_w)])

        lanes = lax.iota(jnp.int32, 16)

        def fire(r, buf):
            iu = idx_u[pl.ds(r, 16)][0]
            il = idx_l[pl.ds(r, 16)][0]
            ou = pl.multiple_of((iu >> 7) * 128, 128)
            ol = pl.multiple_of((il >> 7) * 128, 128)
            pltpu.async_copy(
                tu_hbm.at[pl.ds(0, 32), pl.ds(ou, 128)],
                tiles.at[buf, 0], sems[buf])
            pltpu.async_copy(
                tl_hbm.at[pl.ds(0, 32), pl.ds(ol, 128)],
                tiles.at[buf, 1], sems[buf])

        def drain(buf):
            for _ in range(2):
                pltpu.make_async_copy(
                    tu_hbm.at[pl.ds(0, 32), pl.ds(0, 128)],
                    tiles.at[buf, 0], sems[buf]).wait()

        def extract(r, buf):
            iu = idx_u[pl.ds(r, 16)][0]
            il = idx_l[pl.ds(r, 16)][0]
            imu = jnp.full((16,), iu & 127, jnp.int32)
            iml = jnp.full((16,), il & 127, jnp.int32)
            asm[r, pl.ds(0, 16)] = plsc.load_gather(
                tiles.at[buf, 0], [lanes, imu])
            asm[r, pl.ds(16, 16)] = plsc.load_gather(
                tiles.at[buf, 0], [lanes + 16, imu])
            asm[r, pl.ds(32, 16)] = plsc.load_gather(
                tiles.at[buf, 1], [lanes, iml])
            asm[r, pl.ds(48, 16)] = plsc.load_gather(
                tiles.at[buf, 1], [lanes + 16, iml])

        for p in range(_DEPTH - 1):
            fire(p, p)

        def body(s, _):
            for p in range(_DEPTH):
                r = s * _DEPTH + p
                fire(r + _DEPTH - 1, (p + _DEPTH - 1) % _DEPTH)
                drain(p)
                extract(r, p)
            return _

        n_steps = (b_per_w - (_DEPTH - 1)) // _DEPTH
        lax.fori_loop(0, n_steps, body, 0)
        tail = n_steps * _DEPTH
        for r in range(tail, b_per_w):
            rf = r + _DEPTH - 1
            if rf < b_per_w:
                fire(rf, rf % _DEPTH)
            drain(r % _DEPTH)
            extract(r, r % _DEPTH)
        pltpu.sync_copy(asm, out_hbm.at[pl.ds(base, b_per_w)])

    return k


def kernel(user_fea, table_user, table_location):
    B, _ = user_fea.shape
    D = table_user.shape[1]
    info = plsc.get_sparse_core_info()
    NC, NS = info.num_cores, info.num_subcores
    uid = user_fea[:, 0].astype(jnp.int32)
    lid = user_fea[:, 1].astype(jnp.int32)
    k = _build(B, D, NC, NS)
    out = k(uid, lid, table_user.T, table_location.T)
    return out[:, :2 * D]
